# Initial kernel scaffold; baseline (speedup 1.0000x reference)
import jax
import jax.numpy as jnp
from jax.experimental import pallas as pl


def kernel(global_embedding, geometry_latent, global_min, global_max, geo_embeddings, knn_threshold):
    raise NotImplementedError("write your pallas kernel here")


# speedup vs baseline: 2.8018x; 2.8018x over previous
import jax
import jax.numpy as jnp
from jax.experimental import pallas as pl
from jax.experimental.pallas import tpu as pltpu

_Q = 256
_D = 16
_GD = 128
_N = 100000
_K = 10
_CHUNK = 6400
_NCHUNKS = 16
_NPAD = _CHUNK * _NCHUNKS
_INF = float("inf")


def _oodguard_body(q_ref, s_ref, ge_ref, gmin_ref, gmax_ref, thr_ref,
                   avg_ref, mask_ref, frac_ref, run_ref):
    i = pl.program_id(0)

    @pl.when(i == 0)
    def _init():
        ge = ge_ref[...]
        oob = ((ge < gmin_ref[...]) | (ge > gmax_ref[...])).astype(jnp.float32)
        frac = jnp.sum(oob) * (1.0 / (_Q * _GD))
        frac_ref[...] = jnp.zeros((1, 128), jnp.float32) + frac
        run_ref[...] = jnp.full((_Q, 16), _INF, jnp.float32)

    q = q_ref[...]
    qn = q / (jnp.sqrt(jnp.sum(q * q, axis=1, keepdims=True)) + 1e-8)
    q2 = jnp.sum(qn * qn, axis=1, keepdims=True)

    s = s_ref[...]
    sn = s / (jnp.sqrt(jnp.sum(s * s, axis=0, keepdims=True)) + 1e-8)
    s2 = jnp.sum(sn * sn, axis=0, keepdims=True)

    dots = jnp.dot(qn, sn, preferred_element_type=jnp.float32)
    d2 = jnp.maximum(q2 + s2 - 2.0 * dots, 0.0)

    lane = jax.lax.broadcasted_iota(jnp.int32, (1, _CHUNK), 1)
    d2 = jnp.where(lane < (_N - i * _CHUNK), d2, _INF)

    rem = d2
    run = run_ref[...]
    ms, ccs = [], []
    cum = jnp.zeros((_Q, 1), jnp.float32)
    for _ in range(_K):
        m = jnp.minimum(jnp.min(rem, axis=1, keepdims=True),
                        jnp.min(run, axis=1, keepdims=True))
        eqc = rem == m
        eqr = run == m
        cnt = (jnp.sum(eqc.astype(jnp.float32), axis=1, keepdims=True)
               + jnp.sum(eqr.astype(jnp.float32), axis=1, keepdims=True))
        rem = jnp.where(eqc, _INF, rem)
        run = jnp.where(eqr, _INF, run)
        cum = cum + cnt
        ms.append(m)
        ccs.append(cum)
    mvals = jnp.concatenate(ms, axis=1)
    ccum = jnp.concatenate(ccs, axis=1)
    cols = [jnp.min(jnp.where(ccum > j, mvals, _INF), axis=1, keepdims=True)
            for j in range(_K)]
    cols += [jnp.full((_Q, 1), _INF, jnp.float32)] * (16 - _K)
    new_run = jnp.concatenate(cols, axis=1)
    run_ref[...] = new_run

    @pl.when(i == _NCHUNKS - 1)
    def _fini():
        d = jnp.sqrt(new_run[:, :_K] + 1e-12)
        avg = jnp.sum(d, axis=1, keepdims=True) * (1.0 / _K)
        avg_ref[...] = jnp.broadcast_to(avg, (_Q, 128))
        mask = (avg > thr_ref[0, 0]).astype(jnp.float32)
        mask_ref[...] = jnp.broadcast_to(mask, (_Q, 128))


def kernel(global_embedding, geometry_latent, global_min, global_max,
           geo_embeddings, knn_threshold):
    geo_t = jnp.pad(geo_embeddings, ((0, _NPAD - _N), (0, 0))).T
    gmin = global_min.reshape(1, _GD)
    gmax = global_max.reshape(1, _GD)
    thr = jnp.asarray(knn_threshold, jnp.float32).reshape(1, 1)

    avg_b, mask_b, frac_b = pl.pallas_call(
        _oodguard_body,
        grid=(_NCHUNKS,),
        in_specs=[
            pl.BlockSpec((_Q, _D), lambda i: (0, 0)),
            pl.BlockSpec((_D, _CHUNK), lambda i: (0, i)),
            pl.BlockSpec((_Q, _GD), lambda i: (0, 0)),
            pl.BlockSpec((1, _GD), lambda i: (0, 0)),
            pl.BlockSpec((1, _GD), lambda i: (0, 0)),
            pl.BlockSpec((1, 1), lambda i: (0, 0)),
        ],
        out_specs=[
            pl.BlockSpec((_Q, 128), lambda i: (0, 0)),
            pl.BlockSpec((_Q, 128), lambda i: (0, 0)),
            pl.BlockSpec((1, 128), lambda i: (0, 0)),
        ],
        out_shape=[
            jax.ShapeDtypeStruct((_Q, 128), jnp.float32),
            jax.ShapeDtypeStruct((_Q, 128), jnp.float32),
            jax.ShapeDtypeStruct((1, 128), jnp.float32),
        ],
        scratch_shapes=[pltpu.VMEM((_Q, 16), jnp.float32)],
    )(geometry_latent, geo_t, global_embedding, gmin, gmax, thr)

    avg = avg_b[:, 0]
    ood_mask = mask_b[:, 0].astype(bool)
    frac_oob = frac_b[0, 0]
    return (avg, ood_mask, frac_oob)
